# baseline (speedup 1.0000x reference)
import jax
import jax.numpy as jnp
from jax.experimental import pallas as pl
from jax.experimental.pallas import tpu as pltpu


def _hw_kernel(h_ref, w_ref, o_ref):
    o_ref[...] = jnp.dot(h_ref[...], w_ref[...],
                         preferred_element_type=jnp.float32)


def _ahw_kernel(a_ref, hw_ref, b_ref, o_ref):
    o_ref[...] = jnp.dot(a_ref[...], hw_ref[...],
                         preferred_element_type=jnp.float32) + b_ref[...]


def kernel(structure, H, W, bias):
    n, a_dim = H.shape
    b_dim = W.shape[1]

    hw = pl.pallas_call(
        _hw_kernel,
        out_shape=jax.ShapeDtypeStruct((n, b_dim), jnp.float32),
    )(H, W)

    bm = 200
    out = pl.pallas_call(
        _ahw_kernel,
        grid=(n // bm,),
        in_specs=[
            pl.BlockSpec((bm, n), lambda i: (i, 0)),
            pl.BlockSpec((n, b_dim), lambda i: (0, 0)),
            pl.BlockSpec((1, b_dim), lambda i: (0, 0)),
        ],
        out_specs=pl.BlockSpec((bm, b_dim), lambda i: (i, 0)),
        out_shape=jax.ShapeDtypeStruct((n, b_dim), jnp.float32),
        compiler_params=pltpu.CompilerParams(
            dimension_semantics=("parallel",)),
    )(structure, hw, bias.reshape(1, b_dim))
    return out


# speedup vs baseline: 1.0248x; 1.0248x over previous
import jax
import jax.numpy as jnp
from jax.experimental import pallas as pl
from jax.experimental.pallas import tpu as pltpu

_BM = 200
_NBUF = 4


def _mp_kernel(a_hbm, h_ref, w_ref, b_ref, o_ref, hw_ref, buf_ref, sem):
    n = h_ref.shape[0]
    nchunks = n // _BM

    hw_ref[...] = jnp.dot(h_ref[...], w_ref[...],
                          preferred_element_type=jnp.float32)

    def copy(i, slot):
        return pltpu.make_async_copy(
            a_hbm.at[pl.ds(i * _BM, _BM), :], buf_ref.at[slot], sem.at[slot])

    for i in range(_NBUF):
        copy(i, i).start()

    for i in range(nchunks):
        slot = i % _NBUF
        copy(i, slot).wait()
        o_ref[pl.ds(i * _BM, _BM), :] = jnp.dot(
            buf_ref[slot], hw_ref[...],
            preferred_element_type=jnp.float32) + b_ref[...]
        nxt = i + _NBUF
        if nxt < nchunks:
            copy(nxt, slot).start()


def kernel(structure, H, W, bias):
    n, a_dim = H.shape
    b_dim = W.shape[1]

    out = pl.pallas_call(
        _mp_kernel,
        in_specs=[
            pl.BlockSpec(memory_space=pltpu.MemorySpace.HBM),
            pl.BlockSpec(memory_space=pltpu.MemorySpace.VMEM),
            pl.BlockSpec(memory_space=pltpu.MemorySpace.VMEM),
            pl.BlockSpec(memory_space=pltpu.MemorySpace.VMEM),
        ],
        out_specs=pl.BlockSpec(memory_space=pltpu.MemorySpace.VMEM),
        out_shape=jax.ShapeDtypeStruct((n, b_dim), jnp.float32),
        scratch_shapes=[
            pltpu.VMEM((n, b_dim), jnp.float32),
            pltpu.VMEM((_NBUF, _BM, n), jnp.float32),
            pltpu.SemaphoreType.DMA((_NBUF,)),
        ],
    )(structure, H, W, bias.reshape(1, b_dim))
    return out


# speedup vs baseline: 1.0379x; 1.0127x over previous
import jax
import jax.numpy as jnp
from jax.experimental import pallas as pl
from jax.experimental.pallas import tpu as pltpu


def _fused_kernel(a_ref, h_ref, w_ref, b_ref, o_ref, hw_ref):
    @pl.when(pl.program_id(0) == 0)
    def _():
        hw_ref[...] = jnp.dot(h_ref[...], w_ref[...],
                              preferred_element_type=jnp.float32)

    o_ref[...] = jnp.dot(a_ref[...], hw_ref[...],
                         preferred_element_type=jnp.float32) + b_ref[...]


def kernel(structure, H, W, bias):
    n, a_dim = H.shape
    b_dim = W.shape[1]

    bm = 400
    out = pl.pallas_call(
        _fused_kernel,
        grid=(n // bm,),
        in_specs=[
            pl.BlockSpec((bm, n), lambda i: (i, 0)),
            pl.BlockSpec((n, a_dim), lambda i: (0, 0)),
            pl.BlockSpec((a_dim, b_dim), lambda i: (0, 0)),
            pl.BlockSpec((1, b_dim), lambda i: (0, 0)),
        ],
        out_specs=pl.BlockSpec((bm, b_dim), lambda i: (i, 0)),
        out_shape=jax.ShapeDtypeStruct((n, b_dim), jnp.float32),
        scratch_shapes=[pltpu.VMEM((n, b_dim), jnp.float32)],
    )(structure, H, W, bias.reshape(1, b_dim))
    return out
